# baseline (speedup 1.0000x reference)
import jax
import jax.numpy as jnp
from jax.experimental import pallas as pl
from jax.experimental.pallas import tpu as pltpu

_B, _F, _H, _W, _C = 1, 64, 32, 32, 768
_T = _H * _W
_EPS = 1e-12

_FB = 16
_HB = 16
_NH = _H // _HB
_NFS = _F // _FB


def _body(g_ref, row_ref, col_ref, tte_ref, w_ref, b_ref, out_ref, acc_ref):
    fs = pl.program_id(1)
    hb = pl.program_id(0)
    part = g_ref[0]
    for i in range(1, _FB):
        part = part + g_ref[i]

    @pl.when(fs == 0)
    def _init():
        acc_ref[...] = part

    @pl.when(fs > 0)
    def _accum():
        acc_ref[...] += part

    @pl.when(fs == _NFS - 1)
    def _finish():
        x = acc_ref[...] * (1.0 / _F)
        row = row_ref[pl.ds(hb * _HB, _HB)]
        x = x + row[:, None, :] + col_ref[...][None, :, :]
        x = x + tte_ref[...][None, :, :]
        mu = jnp.mean(x, axis=-1, keepdims=True)
        var = jnp.mean(jnp.square(x - mu), axis=-1, keepdims=True)
        y = (x - mu) * jax.lax.rsqrt(var + _EPS)
        out_ref[...] = y * w_ref[...][None, :, :] + b_ref[...][None, :, :]


def kernel(grid, row_emb, col_emb, token_type_emb, ln_weight, ln_bias):
    g = grid.reshape(_F, _H, _W, _C)
    w2 = ln_weight.reshape(1, _C)
    b2 = ln_bias.reshape(1, _C)
    out = pl.pallas_call(
        _body,
        grid=(_NH, _NFS),
        in_specs=[
            pl.BlockSpec((_FB, _HB, _W, _C), lambda hb, fs: (fs, hb, 0, 0)),
            pl.BlockSpec((_H, _C), lambda hb, fs: (0, 0)),
            pl.BlockSpec((_W, _C), lambda hb, fs: (0, 0)),
            pl.BlockSpec((1, _C), lambda hb, fs: (0, 0)),
            pl.BlockSpec((1, _C), lambda hb, fs: (0, 0)),
            pl.BlockSpec((1, _C), lambda hb, fs: (0, 0)),
        ],
        out_specs=pl.BlockSpec((_HB, _W, _C), lambda hb, fs: (hb, 0, 0)),
        out_shape=jax.ShapeDtypeStruct((_H, _W, _C), jnp.float32),
        scratch_shapes=[pltpu.VMEM((_HB, _W, _C), jnp.float32)],
    )(g, row_emb, col_emb, token_type_emb, w2, b2)
    return out.reshape(_B, _T, _C)


# speedup vs baseline: 1.0369x; 1.0369x over previous
import jax
import jax.numpy as jnp
from jax.experimental import pallas as pl
from jax.experimental.pallas import tpu as pltpu

_B, _F, _H, _W, _C = 1, 64, 32, 32, 768
_T = _H * _W
_EPS = 1e-12

_FB = 8
_HB = 16
_NH = _H // _HB
_NFS = _F // _FB


def _body(g_ref, row_ref, col_ref, tte_ref, w_ref, b_ref, out_ref, acc_ref):
    fs = pl.program_id(1)
    hb = pl.program_id(0)
    part = g_ref[0]
    for i in range(1, _FB):
        part = part + g_ref[i]

    @pl.when(fs == 0)
    def _init():
        acc_ref[...] = part

    @pl.when(fs > 0)
    def _accum():
        acc_ref[...] += part

    @pl.when(fs == _NFS - 1)
    def _finish():
        x = acc_ref[...] * (1.0 / _F)
        row = row_ref[pl.ds(hb * _HB, _HB)]
        x = x + row[:, None, :] + col_ref[...][None, :, :]
        x = x + tte_ref[...][None, :, :]
        mu = jnp.mean(x, axis=-1, keepdims=True)
        var = jnp.mean(jnp.square(x - mu), axis=-1, keepdims=True)
        y = (x - mu) * jax.lax.rsqrt(var + _EPS)
        out_ref[...] = y * w_ref[...][None, :, :] + b_ref[...][None, :, :]


def kernel(grid, row_emb, col_emb, token_type_emb, ln_weight, ln_bias):
    g = grid.reshape(_F, _H, _W, _C)
    w2 = ln_weight.reshape(1, _C)
    b2 = ln_bias.reshape(1, _C)
    out = pl.pallas_call(
        _body,
        grid=(_NH, _NFS),
        in_specs=[
            pl.BlockSpec((_FB, _HB, _W, _C), lambda hb, fs: (fs, hb, 0, 0)),
            pl.BlockSpec((_H, _C), lambda hb, fs: (0, 0)),
            pl.BlockSpec((_W, _C), lambda hb, fs: (0, 0)),
            pl.BlockSpec((1, _C), lambda hb, fs: (0, 0)),
            pl.BlockSpec((1, _C), lambda hb, fs: (0, 0)),
            pl.BlockSpec((1, _C), lambda hb, fs: (0, 0)),
        ],
        out_specs=pl.BlockSpec((_HB, _W, _C), lambda hb, fs: (hb, 0, 0)),
        out_shape=jax.ShapeDtypeStruct((_H, _W, _C), jnp.float32),
        scratch_shapes=[pltpu.VMEM((_HB, _W, _C), jnp.float32)],
    )(g, row_emb, col_emb, token_type_emb, w2, b2)
    return out.reshape(_B, _T, _C)
